# Initial kernel scaffold; baseline (speedup 1.0000x reference)
import functools

import jax
import jax.numpy as jnp
from jax import lax
from jax.experimental import pallas as pl
from jax.experimental.pallas import tpu as pltpu
from jax.experimental.pallas import tpu_sc as plsc

N = 10000
E = 320000
C = 128
NC = 2
NS = 16
NW = NC * NS
NPAD = 10240
RPW = NPAD // NS
CHUNK = 80
EPW = E // NW
NCH = EPW // CHUNK
DPW = 2 * E // NW
DCH = DPW // CHUNK

_MESH = dict(core_axis_name="c", subcore_axis_name="s")


def _zero_1d(ref, n):
    @pl.loop(0, n, step=16)
    def _(i):
        ref[pl.ds(i, 16)] = jnp.zeros((16,), jnp.float32)


def _zero_2d(ref, n):
    @pl.loop(0, n)
    def _(r):
        @pl.loop(0, C, step=16)
        def _(j):
            ref[r, pl.ds(j, 16)] = jnp.zeros((16,), jnp.float32)


def _sc_hist(ei):
    @functools.partial(
        pl.kernel,
        out_type=jax.ShapeDtypeStruct((NC, NPAD), jnp.float32),
        mesh=plsc.VectorSubcoreMesh(**_MESH),
        scratch_types=[
            pltpu.VMEM((CHUNK,), jnp.int32),
            pltpu.VMEM((CHUNK,), jnp.float32),
            pltpu.VMEM((RPW,), jnp.float32),
            pltpu.VMEM_SHARED((NPAD,), jnp.float32),
            pltpu.SemaphoreType.DMA,
        ],
    )
    def k(ei_hbm, out_hbm, idx_v, ones_v, zbuf_v, acc_s, sem):
        cid = lax.axis_index("c")
        sid = lax.axis_index("s")
        wid = cid * NS + sid

        @pl.loop(0, CHUNK, step=16)
        def _(i):
            ones_v[pl.ds(i, 16)] = jnp.full((16,), 1.0, jnp.float32)

        _zero_1d(zbuf_v, RPW)
        pltpu.sync_copy(zbuf_v, acc_s.at[pl.ds(sid * RPW, RPW)])
        plsc.subcore_barrier()

        base = wid * EPW

        @pl.loop(0, NCH)
        def _(ci):
            pltpu.sync_copy(ei_hbm.at[1, pl.ds(base + ci * CHUNK, CHUNK)], idx_v)
            pltpu.sync_copy(ones_v, acc_s.at[idx_v], add=True)

        plsc.subcore_barrier()
        pltpu.sync_copy(acc_s.at[pl.ds(sid * RPW, RPW)],
                        out_hbm.at[cid, pl.ds(sid * RPW, RPW)])

    return k(ei)


def _sc_agg(ei, y):
    @functools.partial(
        pl.kernel,
        out_type=jax.ShapeDtypeStruct((NC, NPAD, C), jnp.float32),
        mesh=plsc.VectorSubcoreMesh(**_MESH),
        scratch_types=[
            pltpu.VMEM((CHUNK,), jnp.int32),
            pltpu.VMEM((CHUNK,), jnp.int32),
            pltpu.VMEM((CHUNK, C), jnp.float32),
            pltpu.VMEM_SHARED((NPAD, C), jnp.float32),
            pltpu.SemaphoreType.DMA,
        ],
    )
    def k(ei_hbm, y_hbm, out_hbm, sidx_v, didx_v, rows_v, acc_s, sem):
        cid = lax.axis_index("c")
        sid = lax.axis_index("s")
        wid = cid * NS + sid

        _zero_2d(rows_v, CHUNK)

        @pl.loop(0, RPW // CHUNK)
        def _(t):
            pltpu.sync_copy(rows_v, acc_s.at[pl.ds(sid * RPW + t * CHUNK, CHUNK)])

        plsc.subcore_barrier()

        base = wid * EPW

        @pl.loop(0, NCH)
        def _(ci):
            off = base + ci * CHUNK
            pltpu.sync_copy(ei_hbm.at[0, pl.ds(off, CHUNK)], sidx_v)
            pltpu.sync_copy(ei_hbm.at[1, pl.ds(off, CHUNK)], didx_v)
            pltpu.async_copy(y_hbm.at[sidx_v], rows_v, sem).wait()
            pltpu.sync_copy(rows_v, acc_s.at[didx_v], add=True)

        plsc.subcore_barrier()
        pltpu.sync_copy(acc_s.at[pl.ds(sid * RPW, RPW)],
                        out_hbm.at[cid, pl.ds(sid * RPW, RPW)])

    return k(ei, y)


def _sc_decode_gather(eli, z):
    @functools.partial(
        pl.kernel,
        out_type=jax.ShapeDtypeStruct((NC, E, C), jnp.float32),
        mesh=plsc.VectorSubcoreMesh(**_MESH),
        scratch_types=[
            pltpu.VMEM((CHUNK,), jnp.int32),
            pltpu.VMEM((CHUNK, C), jnp.float32),
            pltpu.SemaphoreType.DMA,
        ],
    )
    def k(eli_hbm, z_hbm, out_hbm, idx_v, rows_v, sem):
        cid = lax.axis_index("c")
        sid = lax.axis_index("s")

        @pl.loop(0, DCH)
        def _(ci):
            off = sid * DPW + ci * CHUNK
            pltpu.sync_copy(eli_hbm.at[cid, pl.ds(off, CHUNK)], idx_v)
            pltpu.async_copy(z_hbm.at[idx_v], rows_v, sem).wait()
            pltpu.sync_copy(rows_v, out_hbm.at[cid, pl.ds(off, CHUNK)])

    return k(eli, z)


_BM = 1250


def _tc_mm_scale(x, W, h0, h1):
    def body(x_ref, w_ref, h0_ref, h1_ref, y_ref, d_ref):
        d = lax.rsqrt(h0_ref[...] + h1_ref[...] + 1.0)
        y_ref[...] = d * jnp.dot(x_ref[...], w_ref[...],
                                 preferred_element_type=jnp.float32,
                                 precision=lax.Precision.HIGHEST)
        d_ref[...] = d

    return pl.pallas_call(
        body,
        grid=(N // _BM,),
        in_specs=[
            pl.BlockSpec((_BM, C), lambda i: (i, 0)),
            pl.BlockSpec((C, C), lambda i: (0, 0)),
            pl.BlockSpec((_BM, 1), lambda i: (i, 0)),
            pl.BlockSpec((_BM, 1), lambda i: (i, 0)),
        ],
        out_specs=[
            pl.BlockSpec((_BM, C), lambda i: (i, 0)),
            pl.BlockSpec((_BM, 1), lambda i: (i, 0)),
        ],
        out_shape=[
            jax.ShapeDtypeStruct((N, C), jnp.float32),
            jax.ShapeDtypeStruct((N, 1), jnp.float32),
        ],
    )(x, W, h0, h1)


def _tc_fused_mid(acc, y1, dinv, b1, W2):
    def body(a0_ref, a1_ref, y_ref, d_ref, b_ref, w_ref, o_ref):
        d = d_ref[...]
        h = jnp.maximum(d * (a0_ref[0] + a1_ref[0] + y_ref[...]) + b_ref[...], 0.0)
        o_ref[...] = d * jnp.dot(h, w_ref[...],
                                 preferred_element_type=jnp.float32,
                                 precision=lax.Precision.HIGHEST)

    return pl.pallas_call(
        body,
        grid=(N // _BM,),
        in_specs=[
            pl.BlockSpec((1, _BM, C), lambda i: (0, i, 0)),
            pl.BlockSpec((1, _BM, C), lambda i: (1, i, 0)),
            pl.BlockSpec((_BM, C), lambda i: (i, 0)),
            pl.BlockSpec((_BM, 1), lambda i: (i, 0)),
            pl.BlockSpec((1, C), lambda i: (0, 0)),
            pl.BlockSpec((C, C), lambda i: (0, 0)),
        ],
        out_specs=pl.BlockSpec((_BM, C), lambda i: (i, 0)),
        out_shape=jax.ShapeDtypeStruct((N, C), jnp.float32),
    )(acc, acc, y1, dinv, b1, W2)


def _tc_final(acc, y2, dinv, b2):
    def body(a0_ref, a1_ref, y_ref, d_ref, b_ref, o_ref):
        o_ref[...] = d_ref[...] * (a0_ref[0] + a1_ref[0] + y_ref[...]) + b_ref[...]

    return pl.pallas_call(
        body,
        grid=(N // _BM,),
        in_specs=[
            pl.BlockSpec((1, _BM, C), lambda i: (0, i, 0)),
            pl.BlockSpec((1, _BM, C), lambda i: (1, i, 0)),
            pl.BlockSpec((_BM, C), lambda i: (i, 0)),
            pl.BlockSpec((_BM, 1), lambda i: (i, 0)),
            pl.BlockSpec((1, C), lambda i: (0, 0)),
        ],
        out_specs=pl.BlockSpec((_BM, C), lambda i: (i, 0)),
        out_shape=jax.ShapeDtypeStruct((N, C), jnp.float32),
    )(acc, acc, y2, dinv, b2)


_DBM = 2000


def _tc_dot(zg):
    def body(a_ref, b_ref, o_ref):
        o_ref[...] = jnp.sum(a_ref[0] * b_ref[0], axis=1, keepdims=True)

    return pl.pallas_call(
        body,
        grid=(E // _DBM,),
        in_specs=[
            pl.BlockSpec((1, _DBM, C), lambda i: (0, i, 0)),
            pl.BlockSpec((1, _DBM, C), lambda i: (1, i, 0)),
        ],
        out_specs=pl.BlockSpec((_DBM, 1), lambda i: (i, 0)),
        out_shape=jax.ShapeDtypeStruct((E, 1), jnp.float32),
    )(zg, zg)


def kernel(x, edge_index, edge_label_index, W1, b1, W2, b2):
    ei = edge_index.astype(jnp.int32)
    eli = edge_label_index.astype(jnp.int32)

    hist = _sc_hist(ei)
    h0 = hist[0, :N, None]
    h1 = hist[1, :N, None]

    y1, dinv = _tc_mm_scale(x, W1, h0, h1)
    acc1 = _sc_agg(ei, y1)
    y2 = _tc_fused_mid(acc1, y1, dinv, b1.reshape(1, C), W2)
    acc2 = _sc_agg(ei, y2)
    z = _tc_final(acc2, y2, dinv, b2.reshape(1, C))

    zg = _sc_decode_gather(eli, z)
    scores = _tc_dot(zg)
    return scores.reshape(E)


# speedup vs baseline: 7.3917x; 7.3917x over previous
import functools

import jax
import jax.numpy as jnp
from jax import lax
from jax.experimental import pallas as pl
from jax.experimental.pallas import tpu as pltpu
from jax.experimental.pallas import tpu_sc as plsc

N = 10000
E = 320000
C = 128
NC = 2
NS = 16
NW = NC * NS
NPAD = 10240
RPW = NPAD // NS
CHUNK = 80
EPW = E // NW
NCH = EPW // CHUNK
DPW = E // NS
DCH = DPW // CHUNK

_MESH = dict(core_axis_name="c", subcore_axis_name="s")


def _zero_1d(ref, n):
    @pl.loop(0, n, step=16)
    def _(i):
        ref[pl.ds(i, 16)] = jnp.zeros((16,), jnp.float32)


def _zero_2d(ref, n):
    @pl.loop(0, n)
    def _(r):
        @pl.loop(0, C, step=16)
        def _(j):
            ref[r, pl.ds(j, 16)] = jnp.zeros((16,), jnp.float32)


def _sc_hist(dst):
    @functools.partial(
        pl.kernel,
        out_type=[jax.ShapeDtypeStruct((NPAD,), jnp.float32)] * NC,
        mesh=plsc.VectorSubcoreMesh(**_MESH),
        scratch_types=[
            pltpu.VMEM((CHUNK,), jnp.int32),
            pltpu.VMEM((CHUNK,), jnp.float32),
            pltpu.VMEM((RPW,), jnp.float32),
            pltpu.VMEM_SHARED((NPAD,), jnp.float32),
            pltpu.SemaphoreType.DMA,
        ],
    )
    def k(dst_hbm, out0_hbm, out1_hbm, idx_v, ones_v, zbuf_v, acc_s, sem):
        cid = lax.axis_index("c")
        sid = lax.axis_index("s")
        wid = cid * NS + sid

        @pl.loop(0, CHUNK, step=16)
        def _(i):
            ones_v[pl.ds(i, 16)] = jnp.full((16,), 1.0, jnp.float32)

        _zero_1d(zbuf_v, RPW)
        pltpu.sync_copy(zbuf_v, acc_s.at[pl.ds(sid * RPW, RPW)])
        plsc.subcore_barrier()

        base = wid * EPW

        @pl.loop(0, NCH)
        def _(ci):
            pltpu.sync_copy(dst_hbm.at[pl.ds(base + ci * CHUNK, CHUNK)], idx_v)
            pltpu.sync_copy(ones_v, acc_s.at[idx_v], add=True)

        plsc.subcore_barrier()

        @pl.when(cid == 0)
        def _():
            pltpu.sync_copy(acc_s.at[pl.ds(sid * RPW, RPW)],
                            out0_hbm.at[pl.ds(sid * RPW, RPW)])

        @pl.when(cid == 1)
        def _():
            pltpu.sync_copy(acc_s.at[pl.ds(sid * RPW, RPW)],
                            out1_hbm.at[pl.ds(sid * RPW, RPW)])

    return k(dst)


def _sc_agg(src, dst, y):
    @functools.partial(
        pl.kernel,
        out_type=[jax.ShapeDtypeStruct((NPAD, C), jnp.float32)] * NC,
        mesh=plsc.VectorSubcoreMesh(**_MESH),
        scratch_types=[
            pltpu.VMEM((CHUNK,), jnp.int32),
            pltpu.VMEM((CHUNK,), jnp.int32),
            pltpu.VMEM((CHUNK, C), jnp.float32),
            pltpu.VMEM_SHARED((NPAD, C), jnp.float32),
            pltpu.SemaphoreType.DMA,
        ],
    )
    def k(src_hbm, dst_hbm, y_hbm, out0_hbm, out1_hbm,
          sidx_v, didx_v, rows_v, acc_s, sem):
        cid = lax.axis_index("c")
        sid = lax.axis_index("s")
        wid = cid * NS + sid

        _zero_2d(rows_v, CHUNK)

        @pl.loop(0, RPW // CHUNK)
        def _(t):
            pltpu.sync_copy(rows_v, acc_s.at[pl.ds(sid * RPW + t * CHUNK, CHUNK)])

        plsc.subcore_barrier()

        base = wid * EPW

        @pl.loop(0, NCH)
        def _(ci):
            off = base + ci * CHUNK
            pltpu.sync_copy(src_hbm.at[pl.ds(off, CHUNK)], sidx_v)
            pltpu.sync_copy(dst_hbm.at[pl.ds(off, CHUNK)], didx_v)
            pltpu.async_copy(y_hbm.at[sidx_v], rows_v, sem).wait()
            pltpu.sync_copy(rows_v, acc_s.at[didx_v], add=True)

        plsc.subcore_barrier()

        @pl.when(cid == 0)
        def _():
            pltpu.sync_copy(acc_s.at[pl.ds(sid * RPW, RPW)],
                            out0_hbm.at[pl.ds(sid * RPW, RPW)])

        @pl.when(cid == 1)
        def _():
            pltpu.sync_copy(acc_s.at[pl.ds(sid * RPW, RPW)],
                            out1_hbm.at[pl.ds(sid * RPW, RPW)])

    return k(src, dst, y)


def _sc_decode_gather(eli0, eli1, z):
    @functools.partial(
        pl.kernel,
        out_type=[jax.ShapeDtypeStruct((E, C), jnp.float32)] * NC,
        mesh=plsc.VectorSubcoreMesh(**_MESH),
        scratch_types=[
            pltpu.VMEM((CHUNK,), jnp.int32),
            pltpu.VMEM((CHUNK, C), jnp.float32),
            pltpu.SemaphoreType.DMA,
        ],
    )
    def k(eli0_hbm, eli1_hbm, z_hbm, outa_hbm, outb_hbm, idx_v, rows_v, sem):
        cid = lax.axis_index("c")
        sid = lax.axis_index("s")

        @pl.when(cid == 0)
        def _():
            @pl.loop(0, DCH)
            def _(ci):
                off = sid * DPW + ci * CHUNK
                pltpu.sync_copy(eli0_hbm.at[pl.ds(off, CHUNK)], idx_v)
                pltpu.async_copy(z_hbm.at[idx_v], rows_v, sem).wait()
                pltpu.sync_copy(rows_v, outa_hbm.at[pl.ds(off, CHUNK)])

        @pl.when(cid == 1)
        def _():
            @pl.loop(0, DCH)
            def _(ci):
                off = sid * DPW + ci * CHUNK
                pltpu.sync_copy(eli1_hbm.at[pl.ds(off, CHUNK)], idx_v)
                pltpu.async_copy(z_hbm.at[idx_v], rows_v, sem).wait()
                pltpu.sync_copy(rows_v, outb_hbm.at[pl.ds(off, CHUNK)])

    return k(eli0, eli1, z)


_BM = 2000


def _tc_mm_scale(x, W, h0, h1):
    def body(x_ref, w_ref, h0_ref, h1_ref, y_ref, d_ref):
        d = lax.rsqrt(h0_ref[...] + h1_ref[...] + 1.0)
        y_ref[...] = d * jnp.dot(x_ref[...], w_ref[...],
                                 preferred_element_type=jnp.float32,
                                 precision=lax.Precision.HIGHEST)
        d_ref[...] = d

    return pl.pallas_call(
        body,
        grid=(N // _BM,),
        in_specs=[
            pl.BlockSpec((_BM, C), lambda i: (i, 0)),
            pl.BlockSpec((C, C), lambda i: (0, 0)),
            pl.BlockSpec((_BM, 1), lambda i: (i, 0)),
            pl.BlockSpec((_BM, 1), lambda i: (i, 0)),
        ],
        out_specs=[
            pl.BlockSpec((_BM, C), lambda i: (i, 0)),
            pl.BlockSpec((_BM, 1), lambda i: (i, 0)),
        ],
        out_shape=[
            jax.ShapeDtypeStruct((N, C), jnp.float32),
            jax.ShapeDtypeStruct((N, 1), jnp.float32),
        ],
    )(x, W, h0, h1)


def _tc_fused_mid(acc0, acc1, y1, dinv, b1, W2):
    def body(a0_ref, a1_ref, y_ref, d_ref, b_ref, w_ref, o_ref):
        d = d_ref[...]
        h = jnp.maximum(d * (a0_ref[...] + a1_ref[...] + y_ref[...]) + b_ref[...],
                        0.0)
        o_ref[...] = d * jnp.dot(h, w_ref[...],
                                 preferred_element_type=jnp.float32,
                                 precision=lax.Precision.HIGHEST)

    return pl.pallas_call(
        body,
        grid=(N // _BM,),
        in_specs=[
            pl.BlockSpec((_BM, C), lambda i: (i, 0)),
            pl.BlockSpec((_BM, C), lambda i: (i, 0)),
            pl.BlockSpec((_BM, C), lambda i: (i, 0)),
            pl.BlockSpec((_BM, 1), lambda i: (i, 0)),
            pl.BlockSpec((1, C), lambda i: (0, 0)),
            pl.BlockSpec((C, C), lambda i: (0, 0)),
        ],
        out_specs=pl.BlockSpec((_BM, C), lambda i: (i, 0)),
        out_shape=jax.ShapeDtypeStruct((N, C), jnp.float32),
    )(acc0, acc1, y1, dinv, b1, W2)


def _tc_final(acc0, acc1, y2, dinv, b2):
    def body(a0_ref, a1_ref, y_ref, d_ref, b_ref, o_ref):
        o_ref[...] = (d_ref[...] * (a0_ref[...] + a1_ref[...] + y_ref[...])
                      + b_ref[...])

    return pl.pallas_call(
        body,
        grid=(N // _BM,),
        in_specs=[
            pl.BlockSpec((_BM, C), lambda i: (i, 0)),
            pl.BlockSpec((_BM, C), lambda i: (i, 0)),
            pl.BlockSpec((_BM, C), lambda i: (i, 0)),
            pl.BlockSpec((_BM, 1), lambda i: (i, 0)),
            pl.BlockSpec((1, C), lambda i: (0, 0)),
        ],
        out_specs=pl.BlockSpec((_BM, C), lambda i: (i, 0)),
        out_shape=jax.ShapeDtypeStruct((N, C), jnp.float32),
    )(acc0, acc1, y2, dinv, b2)


_DBM = 2000


def _tc_dot(za, zb):
    def body(a_ref, b_ref, o_ref):
        o_ref[...] = jnp.sum(a_ref[...] * b_ref[...], axis=1, keepdims=True)

    return pl.pallas_call(
        body,
        grid=(E // _DBM,),
        in_specs=[
            pl.BlockSpec((_DBM, C), lambda i: (i, 0)),
            pl.BlockSpec((_DBM, C), lambda i: (i, 0)),
        ],
        out_specs=pl.BlockSpec((_DBM, 1), lambda i: (i, 0)),
        out_shape=jax.ShapeDtypeStruct((E, 1), jnp.float32),
    )(za, zb)


def kernel(x, edge_index, edge_label_index, W1, b1, W2, b2):
    ei = edge_index.astype(jnp.int32)
    eli = edge_label_index.astype(jnp.int32)
    src, dst = ei[0], ei[1]
    eli0, eli1 = eli[0], eli[1]

    hist0, hist1 = _sc_hist(dst)
    h0 = hist0[:, None]
    h1 = hist1[:, None]

    y1, dinv = _tc_mm_scale(x, W1, h0, h1)
    a10, a11 = _sc_agg(src, dst, y1)
    y2 = _tc_fused_mid(a10, a11, y1, dinv, b1.reshape(1, C), W2)
    a20, a21 = _sc_agg(src, dst, y2)
    z = _tc_final(a20, a21, y2, dinv, b2.reshape(1, C))

    za, zb = _sc_decode_gather(eli0, eli1, z)
    scores = _tc_dot(za, zb)
    return scores.reshape(E)


# speedup vs baseline: 14.5367x; 1.9666x over previous
import functools

import jax
import jax.numpy as jnp
from jax import lax
from jax.experimental import pallas as pl
from jax.experimental.pallas import tpu as pltpu
from jax.experimental.pallas import tpu_sc as plsc

N = 10000
E = 320000
C = 128
NC = 2
NS = 16
NW = NC * NS
NPAD = 10240
RPW = NPAD // NS
CHUNK = 80
EPW = E // NW
NCH = EPW // CHUNK
DPW = E // NS
DCH = DPW // CHUNK

_MESH = dict(core_axis_name="c", subcore_axis_name="s")


def _zero_1d(ref, n):
    @pl.loop(0, n, step=16)
    def _(i):
        ref[pl.ds(i, 16)] = jnp.zeros((16,), jnp.float32)


def _zero_2d(ref, n):
    @pl.loop(0, n)
    def _(r):
        @pl.loop(0, C, step=16)
        def _(j):
            ref[r, pl.ds(j, 16)] = jnp.zeros((16,), jnp.float32)


def _sc_hist(dst_r):
    @functools.partial(
        pl.kernel,
        out_type=[jax.ShapeDtypeStruct((NPAD,), jnp.float32)] * NC,
        mesh=plsc.VectorSubcoreMesh(**_MESH),
        scratch_types=[
            pltpu.VMEM((NCH, CHUNK), jnp.int32),
            pltpu.VMEM((CHUNK,), jnp.float32),
            pltpu.VMEM((RPW,), jnp.float32),
            pltpu.VMEM_SHARED((NPAD,), jnp.float32),
            pltpu.SemaphoreType.DMA,
        ],
    )
    def k(dst_hbm, out0_hbm, out1_hbm, idx_v, ones_v, zbuf_v, acc_s, sem):
        cid = lax.axis_index("c")
        sid = lax.axis_index("s")
        wid = cid * NS + sid

        pltpu.sync_copy(dst_hbm.at[wid], idx_v)

        @pl.loop(0, CHUNK, step=16)
        def _(i):
            ones_v[pl.ds(i, 16)] = jnp.full((16,), 1.0, jnp.float32)

        _zero_1d(zbuf_v, RPW)
        pltpu.sync_copy(zbuf_v, acc_s.at[pl.ds(sid * RPW, RPW)])
        plsc.subcore_barrier()

        @pl.loop(0, NCH)
        def _(ci):
            pltpu.sync_copy(ones_v, acc_s.at[idx_v.at[ci]], add=True)

        plsc.subcore_barrier()

        @pl.when(cid == 0)
        def _():
            pltpu.sync_copy(acc_s.at[pl.ds(sid * RPW, RPW)],
                            out0_hbm.at[pl.ds(sid * RPW, RPW)])

        @pl.when(cid == 1)
        def _():
            pltpu.sync_copy(acc_s.at[pl.ds(sid * RPW, RPW)],
                            out1_hbm.at[pl.ds(sid * RPW, RPW)])

    return k(dst_r)


_NBUF = 5
_ANB = 4
_NGRP = 31


def _sc_agg(pgrp, st, dt, y):
    @functools.partial(
        pl.kernel,
        out_type=[jax.ShapeDtypeStruct((NPAD, C), jnp.float32)] * NC,
        mesh=plsc.VectorSubcoreMesh(**_MESH),
        scratch_types=[
            pltpu.VMEM((2 * _ANB, CHUNK), jnp.int32),
            pltpu.VMEM((2 * _ANB, CHUNK), jnp.int32),
            pltpu.VMEM((1, CHUNK), jnp.int32),
            pltpu.VMEM((1, CHUNK), jnp.int32),
            pltpu.VMEM((_ANB, CHUNK, C), jnp.float32),
            pltpu.VMEM_SHARED((NPAD, C), jnp.float32),
            pltpu.SemaphoreType.DMA((_ANB,)),
        ],
    )
    def k(p_hbm, st_hbm, dt_hbm, y_hbm, out0_hbm, out1_hbm,
          ixa, ixb, tsx, tdx, rows_v, acc_s, sg):
        cid = lax.axis_index("c")
        sid = lax.axis_index("s")
        wid = cid * NS + sid

        _zero_2d(rows_v.at[0], CHUNK)

        @pl.loop(0, RPW // CHUNK)
        def _(t):
            pltpu.sync_copy(rows_v.at[0],
                            acc_s.at[pl.ds(sid * RPW + t * CHUNK, CHUNK)])

        plsc.subcore_barrier()

        pltpu.sync_copy(p_hbm.at[wid, 0], ixa)
        for b in range(_ANB):
            pltpu.async_copy(y_hbm.at[ixa.at[b]], rows_v.at[b], sg.at[b])

        def halfstep(gcur, cur, nxt):
            pltpu.sync_copy(p_hbm.at[wid, gcur + 1], nxt)
            for b in range(_ANB):
                pltpu.make_async_copy(y_hbm.at[cur.at[b]],
                                      rows_v.at[b], sg.at[b]).wait()
                pltpu.sync_copy(rows_v.at[b], acc_s.at[cur.at[_ANB + b]],
                                add=True)
                pltpu.async_copy(y_hbm.at[nxt.at[b]], rows_v.at[b], sg.at[b])

        @pl.loop(0, (_NGRP - 1) // 2)
        def _(t):
            halfstep(2 * t, ixa, ixb)
            halfstep(2 * t + 1, ixb, ixa)

        for b in range(_ANB):
            pltpu.make_async_copy(y_hbm.at[ixa.at[b]],
                                  rows_v.at[b], sg.at[b]).wait()
            pltpu.sync_copy(rows_v.at[b], acc_s.at[ixa.at[_ANB + b]], add=True)

        pltpu.sync_copy(st_hbm.at[wid], tsx)
        pltpu.sync_copy(dt_hbm.at[wid], tdx)
        pltpu.async_copy(y_hbm.at[tsx.at[0]], rows_v.at[0], sg.at[0]).wait()
        pltpu.sync_copy(rows_v.at[0], acc_s.at[tdx.at[0]], add=True)

        plsc.subcore_barrier()

        @pl.when(cid == 0)
        def _():
            pltpu.sync_copy(acc_s.at[pl.ds(sid * RPW, RPW)],
                            out0_hbm.at[pl.ds(sid * RPW, RPW)])

        @pl.when(cid == 1)
        def _():
            pltpu.sync_copy(acc_s.at[pl.ds(sid * RPW, RPW)],
                            out1_hbm.at[pl.ds(sid * RPW, RPW)])

    return k(pgrp, st, dt, y)


def _sc_decode_gather(eli0_r, eli1_r, z):
    @functools.partial(
        pl.kernel,
        out_type=[jax.ShapeDtypeStruct((E, C), jnp.float32)] * NC,
        mesh=plsc.VectorSubcoreMesh(**_MESH),
        scratch_types=[
            pltpu.VMEM((DCH, CHUNK), jnp.int32),
            pltpu.VMEM((_NBUF, CHUNK, C), jnp.float32),
            pltpu.SemaphoreType.DMA((_NBUF,)),
        ],
    )
    def k(eli0_hbm, eli1_hbm, z_hbm, outa_hbm, outb_hbm, idx_v, rows_v, sg):
        cid = lax.axis_index("c")
        sid = lax.axis_index("s")

        def side(e_hbm, o_hbm):
            pltpu.sync_copy(e_hbm.at[sid], idx_v)
            for b in range(_NBUF):
                pltpu.async_copy(z_hbm.at[idx_v.at[b]], rows_v.at[b], sg.at[b])

            @pl.loop(0, DCH // _NBUF)
            def _(g):
                for b in range(_NBUF):
                    c = g * _NBUF + b
                    pltpu.make_async_copy(z_hbm.at[idx_v.at[c]],
                                          rows_v.at[b], sg.at[b]).wait()
                    pltpu.sync_copy(
                        rows_v.at[b],
                        o_hbm.at[pl.ds(sid * DPW + c * CHUNK, CHUNK)])

                    @pl.when(g < DCH // _NBUF - 1)
                    def _():
                        pltpu.async_copy(z_hbm.at[idx_v.at[c + _NBUF]],
                                         rows_v.at[b], sg.at[b])

        @pl.when(cid == 0)
        def _():
            side(eli0_hbm, outa_hbm)

        @pl.when(cid == 1)
        def _():
            side(eli1_hbm, outb_hbm)

    return k(eli0_r, eli1_r, z)


_BM = 2000


def _tc_mm_scale(x, W, h0, h1):
    def body(x_ref, w_ref, h0_ref, h1_ref, y_ref, d_ref):
        d = lax.rsqrt(h0_ref[...] + h1_ref[...] + 1.0)
        y_ref[...] = d * jnp.dot(x_ref[...], w_ref[...],
                                 preferred_element_type=jnp.float32,
                                 precision=lax.Precision.HIGHEST)
        d_ref[...] = d

    return pl.pallas_call(
        body,
        grid=(N // _BM,),
        in_specs=[
            pl.BlockSpec((_BM, C), lambda i: (i, 0)),
            pl.BlockSpec((C, C), lambda i: (0, 0)),
            pl.BlockSpec((_BM, 1), lambda i: (i, 0)),
            pl.BlockSpec((_BM, 1), lambda i: (i, 0)),
        ],
        out_specs=[
            pl.BlockSpec((_BM, C), lambda i: (i, 0)),
            pl.BlockSpec((_BM, 1), lambda i: (i, 0)),
        ],
        out_shape=[
            jax.ShapeDtypeStruct((N, C), jnp.float32),
            jax.ShapeDtypeStruct((N, 1), jnp.float32),
        ],
    )(x, W, h0, h1)


def _tc_fused_mid(acc0, acc1, y1, dinv, b1, W2):
    def body(a0_ref, a1_ref, y_ref, d_ref, b_ref, w_ref, o_ref):
        d = d_ref[...]
        h = jnp.maximum(d * (a0_ref[...] + a1_ref[...] + y_ref[...]) + b_ref[...],
                        0.0)
        o_ref[...] = d * jnp.dot(h, w_ref[...],
                                 preferred_element_type=jnp.float32,
                                 precision=lax.Precision.HIGHEST)

    return pl.pallas_call(
        body,
        grid=(N // _BM,),
        in_specs=[
            pl.BlockSpec((_BM, C), lambda i: (i, 0)),
            pl.BlockSpec((_BM, C), lambda i: (i, 0)),
            pl.BlockSpec((_BM, C), lambda i: (i, 0)),
            pl.BlockSpec((_BM, 1), lambda i: (i, 0)),
            pl.BlockSpec((1, C), lambda i: (0, 0)),
            pl.BlockSpec((C, C), lambda i: (0, 0)),
        ],
        out_specs=pl.BlockSpec((_BM, C), lambda i: (i, 0)),
        out_shape=jax.ShapeDtypeStruct((N, C), jnp.float32),
    )(acc0, acc1, y1, dinv, b1, W2)


def _tc_final(acc0, acc1, y2, dinv, b2):
    def body(a0_ref, a1_ref, y_ref, d_ref, b_ref, o_ref):
        o_ref[...] = (d_ref[...] * (a0_ref[...] + a1_ref[...] + y_ref[...])
                      + b_ref[...])

    return pl.pallas_call(
        body,
        grid=(N // _BM,),
        in_specs=[
            pl.BlockSpec((_BM, C), lambda i: (i, 0)),
            pl.BlockSpec((_BM, C), lambda i: (i, 0)),
            pl.BlockSpec((_BM, C), lambda i: (i, 0)),
            pl.BlockSpec((_BM, 1), lambda i: (i, 0)),
            pl.BlockSpec((1, C), lambda i: (0, 0)),
        ],
        out_specs=pl.BlockSpec((_BM, C), lambda i: (i, 0)),
        out_shape=jax.ShapeDtypeStruct((N, C), jnp.float32),
    )(acc0, acc1, y2, dinv, b2)


_DBM = 2000


def _tc_dot(za, zb):
    def body(a_ref, b_ref, o_ref):
        o_ref[...] = jnp.sum(a_ref[...] * b_ref[...], axis=1, keepdims=True)

    return pl.pallas_call(
        body,
        grid=(E // _DBM,),
        in_specs=[
            pl.BlockSpec((_DBM, C), lambda i: (i, 0)),
            pl.BlockSpec((_DBM, C), lambda i: (i, 0)),
        ],
        out_specs=pl.BlockSpec((_DBM, 1), lambda i: (i, 0)),
        out_shape=jax.ShapeDtypeStruct((E, 1), jnp.float32),
    )(za, zb)


def kernel(x, edge_index, edge_label_index, W1, b1, W2, b2):
    ei = edge_index.astype(jnp.int32)
    eli = edge_label_index.astype(jnp.int32)
    s_r = ei[0].reshape(NW, NCH, CHUNK)
    d_r = ei[1].reshape(NW, NCH, CHUNK)
    sgrp = s_r[:, : _NGRP * _ANB].reshape(NW, _NGRP, _ANB, CHUNK)
    dgrp = d_r[:, : _NGRP * _ANB].reshape(NW, _NGRP, _ANB, CHUNK)
    pgrp = jnp.concatenate([sgrp, dgrp], axis=2)
    st = s_r[:, _NGRP * _ANB :]
    dt = d_r[:, _NGRP * _ANB :]
    eli0 = eli[0].reshape(NS, DCH, CHUNK)
    eli1 = eli[1].reshape(NS, DCH, CHUNK)

    hist0, hist1 = _sc_hist(d_r)
    h0 = hist0[:, None]
    h1 = hist1[:, None]

    y1, dinv = _tc_mm_scale(x, W1, h0, h1)
    a10, a11 = _sc_agg(pgrp, st, dt, y1)
    y2 = _tc_fused_mid(a10, a11, y1, dinv, b1.reshape(1, C), W2)
    a20, a21 = _sc_agg(pgrp, st, dt, y2)
    z = _tc_final(a20, a21, y2, dinv, b2.reshape(1, C))

    za, zb = _sc_decode_gather(eli0, eli1, z)
    scores = _tc_dot(za, zb)
    return scores.reshape(E)
